# baseline (speedup 1.0000x reference)
import jax
import jax.numpy as jnp
from jax import lax
from jax.experimental import pallas as pl
from jax.experimental.pallas import tpu as pltpu
from jax.experimental.pallas import tpu_sc as plsc

B, N = 32, 32
NODE_F, EDGE_F, MSG, PASSES, OUT_F = 128, 16, 128, 3, 128
BN = B * N
E_ROWS = B * N * N
N_WORKERS = 32
PAIRS_PER_W = BN // N_WORKERS
EDGE_ROWS_PER_W = E_ROWS // N_WORKERS


N_CHUNKS = 8
CH = EDGE_ROWS_PER_W // N_CHUNKS
PAIRS_PER_CH = CH // N


def _ea_sc_kernel(a_hbm, e_hbm, out_hbm, av, ev0, ev1, acc_buf, sem0, sem1):
    wid = lax.axis_index("s") * 2 + lax.axis_index("c")
    pltpu.sync_copy(a_hbm.at[pl.ds(wid * PAIRS_PER_W, PAIRS_PER_W), :], av)

    base = wid * EDGE_ROWS_PER_W
    evs = (ev0, ev1)
    sems = (sem0, sem1)
    pltpu.make_async_copy(e_hbm.at[pl.ds(base, CH), :], ev0, sem0).start()
    zeros16 = jnp.zeros((EDGE_F,), jnp.float32)
    for c in range(N_CHUNKS):
        if c + 1 < N_CHUNKS:
            pltpu.make_async_copy(e_hbm.at[pl.ds(base + (c + 1) * CH, CH), :],
                                  evs[(c + 1) % 2], sems[(c + 1) % 2]).start()
        pltpu.make_async_copy(e_hbm.at[pl.ds(base + c * CH, CH), :],
                              evs[c % 2], sems[c % 2]).wait()
        rbuf = evs[c % 2]

        def pair_body(pp, _, c=c, rbuf=rbuf):
            p = c * PAIRS_PER_CH + pp
            a0 = av[p, pl.ds(0, EDGE_F)]
            a1 = av[p, pl.ds(EDGE_F, EDGE_F)]
            acc = zeros16
            for h in range(N):
                s = a0[h] if h < EDGE_F else a1[h - EDGE_F]
                acc = acc + s * rbuf[pp * N + h, :]
            acc_buf[p, pl.ds(0, EDGE_F)] = acc
            for k in range(1, NODE_F // EDGE_F):
                acc_buf[p, pl.ds(k * EDGE_F, EDGE_F)] = zeros16
            return _

        lax.fori_loop(0, PAIRS_PER_CH, pair_body, None)

    pltpu.sync_copy(acc_buf, out_hbm.at[pl.ds(wid * PAIRS_PER_W, PAIRS_PER_W), :])


def _ea_sparsecore(adjacency2d, edges2d):
    mesh = plsc.VectorSubcoreMesh(core_axis_name="c", subcore_axis_name="s")
    return pl.kernel(
        _ea_sc_kernel,
        out_type=jax.ShapeDtypeStruct((BN, NODE_F), jnp.float32),
        mesh=mesh,
        scratch_types=[
            pltpu.MemorySpace.VMEM((PAIRS_PER_W, N), jnp.float32),
            pltpu.MemorySpace.VMEM((CH, EDGE_F), jnp.float32),
            pltpu.MemorySpace.VMEM((CH, EDGE_F), jnp.float32),
            pltpu.MemorySpace.VMEM((PAIRS_PER_W, NODE_F), jnp.float32),
            pltpu.SemaphoreType.DMA,
            pltpu.SemaphoreType.DMA,
        ],
    )(adjacency2d, edges2d)


def _mpnn_kernel(a_ref, h_ref, ea_ref, wmsg_ref, wupd_ref, wout_ref, out_ref):
    Af = a_ref[:]
    H0 = h_ref[:]
    EApad = ea_ref[:]
    W1 = wmsg_ref[0:NODE_F, :]
    W2 = wmsg_ref[NODE_F:2 * NODE_F, :]
    W3 = wmsg_ref[2 * NODE_F:, :]
    Wu1 = wupd_ref[0:NODE_F, :]
    Wu2 = wupd_ref[NODE_F:, :]
    Wo1 = wout_ref[0:NODE_F, :]
    Wo2 = wout_ref[NODE_F:, :]

    f32 = jnp.float32
    W3ext = jnp.concatenate(
        [W3, jnp.zeros((NODE_F - EDGE_F, MSG), f32)], axis=0)
    E3 = jnp.dot(EApad, W3ext, preferred_element_type=f32)

    deg = jnp.sum(Af, axis=1, keepdims=True)
    maskb = deg > 0.0
    maskf = maskb.astype(f32)
    A3 = Af.reshape(B, N, N)

    H = H0
    for _ in range(PASSES):
        Hb = H.reshape(B, N, NODE_F)
        neigh = lax.dot_general(
            A3, Hb, (((2,), (1,)), ((0,), (0,))),
            preferred_element_type=f32).reshape(BN, NODE_F)
        msg = deg * jnp.dot(H, W1, preferred_element_type=f32) \
            + jnp.dot(neigh, W2, preferred_element_type=f32) + E3
        new = jnp.tanh(jnp.dot(H, Wu1, preferred_element_type=f32)
                       + jnp.dot(msg, Wu2, preferred_element_type=f32))
        H = jnp.where(maskb, new, H)

    G1 = jnp.sum((H * maskf).reshape(B, N, NODE_F), axis=1)
    G2 = jnp.sum((H0 * maskf).reshape(B, N, NODE_F), axis=1)
    out_ref[:] = (jnp.dot(G1, Wo1, preferred_element_type=f32)
                  + jnp.dot(G2, Wo2, preferred_element_type=f32))


def kernel(adjacency, nodes, edges, W_msg, W_upd, W_out):
    a2 = adjacency.reshape(BN, N)
    eapad = _ea_sparsecore(a2, edges.reshape(E_ROWS, EDGE_F))
    return pl.pallas_call(
        _mpnn_kernel,
        out_shape=jax.ShapeDtypeStruct((B, OUT_F), jnp.float32),
    )(a2, nodes.reshape(BN, NODE_F), eapad, W_msg, W_upd, W_out)


# speedup vs baseline: 1.0019x; 1.0019x over previous
import jax
import jax.numpy as jnp
from jax import lax
from jax.experimental import pallas as pl
from jax.experimental.pallas import tpu as pltpu
from jax.experimental.pallas import tpu_sc as plsc

B, N = 32, 32
NODE_F, EDGE_F, MSG, PASSES, OUT_F = 128, 16, 128, 3, 128
BN = B * N
E_ROWS = B * N * N
N_WORKERS = 32
PAIRS_PER_W = BN // N_WORKERS
EDGE_ROWS_PER_W = E_ROWS // N_WORKERS


N_CHUNKS = 8
CH = EDGE_ROWS_PER_W // N_CHUNKS
PAIRS_PER_CH = CH // N


def _ea_sc_kernel(a_hbm, e_hbm, out_hbm, av, ev0, ev1, acc_buf, sem0, sem1):
    wid = lax.axis_index("s") * 2 + lax.axis_index("c")
    pltpu.sync_copy(a_hbm.at[pl.ds(wid * PAIRS_PER_W, PAIRS_PER_W), :], av)

    base = wid * EDGE_ROWS_PER_W
    evs = (ev0, ev1)
    sems = (sem0, sem1)
    pltpu.make_async_copy(e_hbm.at[pl.ds(base, CH), :], ev0, sem0).start()
    zeros16 = jnp.zeros((EDGE_F,), jnp.float32)
    for c in range(N_CHUNKS):
        if c + 1 < N_CHUNKS:
            pltpu.make_async_copy(e_hbm.at[pl.ds(base + (c + 1) * CH, CH), :],
                                  evs[(c + 1) % 2], sems[(c + 1) % 2]).start()
        pltpu.make_async_copy(e_hbm.at[pl.ds(base + c * CH, CH), :],
                              evs[c % 2], sems[c % 2]).wait()
        rbuf = evs[c % 2]

        def pair_body(pp, _, c=c, rbuf=rbuf):
            p = c * PAIRS_PER_CH + pp
            a0 = av[p, pl.ds(0, EDGE_F)]
            a1 = av[p, pl.ds(EDGE_F, EDGE_F)]
            acc = zeros16
            dnums = lax.GatherDimensionNumbers(
                offset_dims=(), collapsed_slice_dims=(0,), start_index_map=(0,))
            for h in range(N):
                src = a0 if h < EDGE_F else a1
                w = lax.gather(
                    src, jnp.full((EDGE_F, 1), h % EDGE_F, jnp.int32), dnums,
                    slice_sizes=(1,),
                    mode=lax.GatherScatterMode.PROMISE_IN_BOUNDS)
                acc = acc + w * rbuf[pp * N + h, :]
            acc_buf[p, pl.ds(0, EDGE_F)] = acc
            for k in range(1, NODE_F // EDGE_F):
                acc_buf[p, pl.ds(k * EDGE_F, EDGE_F)] = zeros16
            return _

        lax.fori_loop(0, PAIRS_PER_CH, pair_body, None)

    pltpu.sync_copy(acc_buf, out_hbm.at[pl.ds(wid * PAIRS_PER_W, PAIRS_PER_W), :])


def _ea_sparsecore(adjacency2d, edges2d):
    mesh = plsc.VectorSubcoreMesh(core_axis_name="c", subcore_axis_name="s")
    return pl.kernel(
        _ea_sc_kernel,
        out_type=jax.ShapeDtypeStruct((BN, NODE_F), jnp.float32),
        mesh=mesh,
        scratch_types=[
            pltpu.MemorySpace.VMEM((PAIRS_PER_W, N), jnp.float32),
            pltpu.MemorySpace.VMEM((CH, EDGE_F), jnp.float32),
            pltpu.MemorySpace.VMEM((CH, EDGE_F), jnp.float32),
            pltpu.MemorySpace.VMEM((PAIRS_PER_W, NODE_F), jnp.float32),
            pltpu.SemaphoreType.DMA,
            pltpu.SemaphoreType.DMA,
        ],
        compiler_params=pltpu.CompilerParams(use_tc_tiling_on_sc=True),
    )(adjacency2d, edges2d)


def _mpnn_kernel(a_ref, h_ref, ea_ref, wmsg_ref, wupd_ref, wout_ref, out_ref):
    Af = a_ref[:]
    H0 = h_ref[:]
    EApad = ea_ref[:]
    W1 = wmsg_ref[0:NODE_F, :]
    W2 = wmsg_ref[NODE_F:2 * NODE_F, :]
    W3 = wmsg_ref[2 * NODE_F:, :]
    Wu1 = wupd_ref[0:NODE_F, :]
    Wu2 = wupd_ref[NODE_F:, :]
    Wo1 = wout_ref[0:NODE_F, :]
    Wo2 = wout_ref[NODE_F:, :]

    f32 = jnp.float32
    W3ext = jnp.concatenate(
        [W3, jnp.zeros((NODE_F - EDGE_F, MSG), f32)], axis=0)
    E3 = jnp.dot(EApad, W3ext, preferred_element_type=f32)

    deg = jnp.sum(Af, axis=1, keepdims=True)
    maskb = deg > 0.0
    maskf = maskb.astype(f32)
    A3 = Af.reshape(B, N, N)

    H = H0
    for _ in range(PASSES):
        Hb = H.reshape(B, N, NODE_F)
        neigh = lax.dot_general(
            A3, Hb, (((2,), (1,)), ((0,), (0,))),
            preferred_element_type=f32).reshape(BN, NODE_F)
        msg = deg * jnp.dot(H, W1, preferred_element_type=f32) \
            + jnp.dot(neigh, W2, preferred_element_type=f32) + E3
        new = jnp.tanh(jnp.dot(H, Wu1, preferred_element_type=f32)
                       + jnp.dot(msg, Wu2, preferred_element_type=f32))
        H = jnp.where(maskb, new, H)

    G1 = jnp.sum((H * maskf).reshape(B, N, NODE_F), axis=1)
    G2 = jnp.sum((H0 * maskf).reshape(B, N, NODE_F), axis=1)
    out_ref[:] = (jnp.dot(G1, Wo1, preferred_element_type=f32)
                  + jnp.dot(G2, Wo2, preferred_element_type=f32))


def kernel(adjacency, nodes, edges, W_msg, W_upd, W_out):
    a2 = adjacency.reshape(BN, N)
    eapad = _ea_sparsecore(a2, edges.reshape(E_ROWS, EDGE_F))
    return pl.pallas_call(
        _mpnn_kernel,
        out_shape=jax.ShapeDtypeStruct((B, OUT_F), jnp.float32),
    )(a2, nodes.reshape(BN, NODE_F), eapad, W_msg, W_upd, W_out)


# speedup vs baseline: 1.7104x; 1.7071x over previous
import jax
import jax.numpy as jnp
from jax.experimental import pallas as pl

B, N = 32, 32
NODE_F, EDGE_F, MSG, PASSES, OUT_F = 128, 16, 128, 3, 128
BN = B * N
NEF = N * EDGE_F


def _mpnn_kernel(a_ref, h_ref, er_ref, wmsg_ref, wupd_ref, wout_ref, out_ref):
    Af = a_ref[:]
    H0 = h_ref[:]
    Er = er_ref[:]
    W1 = wmsg_ref[0:NODE_F, :]
    W2 = wmsg_ref[NODE_F:2 * NODE_F, :]
    W3 = wmsg_ref[2 * NODE_F:, :]
    Wu1 = wupd_ref[0:NODE_F, :]
    Wu2 = wupd_ref[NODE_F:, :]
    Wo1 = wout_ref[0:NODE_F, :]
    Wo2 = wout_ref[NODE_F:, :]

    f32 = jnp.float32
    r_row = jax.lax.broadcasted_iota(jnp.int32, (N, NEF), 0)
    r_col = jax.lax.broadcasted_iota(jnp.int32, (N, NEF), 1)
    R = (r_col // EDGE_F == r_row).astype(f32)
    t_row = jax.lax.broadcasted_iota(jnp.int32, (NEF, EDGE_F), 0)
    t_col = jax.lax.broadcasted_iota(jnp.int32, (NEF, EDGE_F), 1)
    Rt = (t_row % EDGE_F == t_col).astype(f32)

    Aexp = jnp.dot(Af, R, preferred_element_type=f32)
    EA = jnp.dot(Aexp * Er, Rt, preferred_element_type=f32)
    E3 = jnp.dot(EA, W3, preferred_element_type=f32)

    deg = jnp.sum(Af, axis=1, keepdims=True)
    maskb = deg > 0.0
    maskf = maskb.astype(f32)
    A3 = Af.reshape(B, N, N)

    H = H0
    for _ in range(PASSES):
        Hb = H.reshape(B, N, NODE_F)
        neigh = jax.lax.dot_general(
            A3, Hb, (((2,), (1,)), ((0,), (0,))),
            preferred_element_type=f32).reshape(BN, NODE_F)
        msg = deg * jnp.dot(H, W1, preferred_element_type=f32) \
            + jnp.dot(neigh, W2, preferred_element_type=f32) + E3
        new = jnp.tanh(jnp.dot(H, Wu1, preferred_element_type=f32)
                       + jnp.dot(msg, Wu2, preferred_element_type=f32))
        H = jnp.where(maskb, new, H)

    G1 = jnp.sum((H * maskf).reshape(B, N, NODE_F), axis=1)
    G2 = jnp.sum((H0 * maskf).reshape(B, N, NODE_F), axis=1)
    out_ref[:] = (jnp.dot(G1, Wo1, preferred_element_type=f32)
                  + jnp.dot(G2, Wo2, preferred_element_type=f32))


def kernel(adjacency, nodes, edges, W_msg, W_upd, W_out):
    return pl.pallas_call(
        _mpnn_kernel,
        out_shape=jax.ShapeDtypeStruct((B, OUT_F), jnp.float32),
    )(adjacency.reshape(BN, N), nodes.reshape(BN, NODE_F),
      edges.reshape(BN, NEF), W_msg, W_upd, W_out)
